# baseline (speedup 1.0000x reference)
import jax
import jax.numpy as jnp
from jax.experimental import pallas as pl
from jax.experimental.pallas import tpu as pltpu


def _pos_kernel(row_ref, col_ref, out_ref, scratch):
    _, two_d, h, w = out_ref.shape
    d = two_d // 2

    @pl.when(pl.program_id(0) == 0)
    def _():
        ceT = jnp.transpose(col_ref[:, :])
        reT = jnp.transpose(row_ref[:, :])
        scratch[:d] = jnp.broadcast_to(ceT[:, None, :], (d, h, w))
        scratch[d:] = jnp.broadcast_to(reT[:, :, None], (d, h, w))

    out_ref[0] = scratch[:, :, :]


def kernel(x, row_embed, col_embed):
    b = x.shape[0]
    h, w = x.shape[-2], x.shape[-1]
    d = row_embed.shape[-1]
    return pl.pallas_call(
        _pos_kernel,
        grid=(b,),
        in_specs=[
            pl.BlockSpec((h, d), lambda i: (0, 0)),
            pl.BlockSpec((w, d), lambda i: (0, 0)),
        ],
        out_specs=pl.BlockSpec((1, 2 * d, h, w), lambda i: (i, 0, 0, 0)),
        out_shape=jax.ShapeDtypeStruct((b, 2 * d, h, w), row_embed.dtype),
        scratch_shapes=[pltpu.VMEM((2 * d, h, w), row_embed.dtype)],
    )(row_embed[:h], col_embed[:w])


# speedup vs baseline: 3.0894x; 3.0894x over previous
import jax
import jax.numpy as jnp
from jax.experimental import pallas as pl
from jax.experimental.pallas import tpu as pltpu


def _pos_kernel(row_ref, col_ref, out_ref, scratch, sems):
    b, rows, lanes = out_ref.shape
    h = row_ref.shape[0]
    w = col_ref.shape[0]
    d = row_ref.shape[1]
    ceT = jnp.transpose(col_ref[:, :])
    reT = jnp.transpose(row_ref[:, :])
    top = jnp.broadcast_to(ceT[:, None, :], (d, h, w)).reshape(d, h * w)
    bot = jnp.broadcast_to(reT[:, :, None], (d, h, w)).reshape(d, h * w)
    scratch[: rows // 2] = top
    scratch[rows // 2 :] = bot
    for i in range(b):
        pltpu.make_async_copy(scratch, out_ref.at[i], sems.at[i]).start(
            priority=i % 2)
    for i in range(b):
        pltpu.make_async_copy(scratch, out_ref.at[i], sems.at[i]).wait()


def kernel(x, row_embed, col_embed):
    b = x.shape[0]
    h, w = x.shape[-2], x.shape[-1]
    d = row_embed.shape[-1]
    rows = 2 * d
    out = pl.pallas_call(
        _pos_kernel,
        in_specs=[
            pl.BlockSpec((h, d), lambda: (0, 0)),
            pl.BlockSpec((w, d), lambda: (0, 0)),
        ],
        out_specs=pl.BlockSpec(memory_space=pl.ANY),
        out_shape=jax.ShapeDtypeStruct((b, rows, h * w), row_embed.dtype),
        scratch_shapes=[
            pltpu.VMEM((rows, h * w), row_embed.dtype),
            pltpu.SemaphoreType.DMA((b,)),
        ],
    )(row_embed[:h], col_embed[:w])
    return out.reshape(b, 2 * d, h, w)


# speedup vs baseline: 10.2715x; 3.3248x over previous
import jax
import jax.numpy as jnp
from jax.experimental import pallas as pl
from jax.experimental.pallas import tpu as pltpu


def _pos_kernel(row_ref, col_ref, out_ref, scratch, sems):
    b = out_ref.shape[0]
    h, d = row_ref.shape
    w = col_ref.shape[0]
    ce = col_ref[:, :]
    re = row_ref[:, :]
    scratch[:, :, :d] = jnp.broadcast_to(ce[None, :, :], (h, w, d))
    scratch[:, :, d:] = jnp.broadcast_to(re[:, None, :], (h, w, d))
    for i in range(b):
        pltpu.make_async_copy(scratch, out_ref.at[i], sems.at[i]).start()
    for i in range(b):
        pltpu.make_async_copy(scratch, out_ref.at[i], sems.at[i]).wait()


def kernel(x, row_embed, col_embed):
    b = x.shape[0]
    h, w = x.shape[-2], x.shape[-1]
    d = row_embed.shape[-1]
    out = pl.pallas_call(
        _pos_kernel,
        in_specs=[
            pl.BlockSpec((h, d), lambda: (0, 0)),
            pl.BlockSpec((w, d), lambda: (0, 0)),
        ],
        out_specs=pl.BlockSpec(memory_space=pl.ANY),
        out_shape=jax.ShapeDtypeStruct((b, h, w, 2 * d), row_embed.dtype),
        scratch_shapes=[
            pltpu.VMEM((h, w, 2 * d), row_embed.dtype),
            pltpu.SemaphoreType.DMA((b,)),
        ],
    )(row_embed[:h], col_embed[:w])
    return jnp.transpose(out, (0, 3, 1, 2))
